# baseline (speedup 1.0000x reference)
import functools

import numpy as np
import jax
import jax.numpy as jnp
from jax import lax
from jax.experimental import pallas as pl
from jax.experimental.pallas import tpu as pltpu
from jax.experimental.pallas import tpu_sc as plsc

_H = 336
_NG = 24
_SCALE = _H // _NG


def _support_matrix() -> np.ndarray:
    i = np.arange(_NG)[:, None]
    j = np.arange(_H)[None, :]
    return (np.abs(j - (_SCALE * i + (_SCALE / 2 - 0.5))) < _SCALE).astype(np.float32)


_SUP = _support_matrix()
_NSTREAM = 4
_INV_SQRT2 = float(1.0 / np.sqrt(2.0))


def _weights_body(m0_ref, m1_ref, sup_ref, supt_ref, w_ref, *, n_ann):
    half = n_ann // 2
    for g, mask_ref in enumerate((m0_ref, m1_ref)):
        m = mask_ref[...].astype(jnp.float32)
        mf = m.reshape(half * _H, _H)
        colp = jax.lax.dot_general(
            mf, supt_ref[...], (((1,), (0,)), ((), ())),
            preferred_element_type=jnp.float32)
        for a in range(half):
            t = colp[a * _H:(a + 1) * _H]
            q = jax.lax.dot_general(
                sup_ref[...], t, (((1,), (0,)), ((), ())),
                preferred_element_type=jnp.float32)
            b = (q > 0).astype(jnp.float32)
            w_ref[g * half + a] = b * (1.0 / (jnp.sum(b) + 1e-8))


def _pool_body(idx_ref, w_ref, *refs):
    del idx_ref
    out_ref = refs[-1]
    for k, f_ref in enumerate(refs[:-1]):
        w = w_ref[k]
        f = f_ref[0]
        out_ref[k] = jax.lax.dot_general(
            w, f, (((1,), (0,)), ((), ())),
            preferred_element_type=jnp.float32)


def _sc_pool_factory(sc_n, tok, d_mm):
    mesh = plsc.VectorSubcoreMesh(core_axis_name="c", subcore_axis_name="s")
    n_per_core = sc_n // 2
    tpt = tok // 16

    @functools.partial(
        pl.kernel, mesh=mesh,
        out_type=jax.ShapeDtypeStruct((sc_n, d_mm), jnp.float32),
        scratch_types=[
            pltpu.VMEM((tpt, d_mm), jnp.float32),
            pltpu.VMEM((n_per_core, d_mm), jnp.float32),
            pltpu.VMEM((tok + 16,), jnp.float32),
            pltpu.VMEM((sc_n + 16,), jnp.int32),
            pltpu.VMEM((16, d_mm), jnp.float32),
            pltpu.VMEM((d_mm,), jnp.float32),
            pltpu.VMEM_SHARED((16, n_per_core, d_mm), jnp.float32),
        ],
    )
    def sc_pool(feats_hbm, wn_hbm, idx_hbm, out_hbm,
                slab, locmf, w_vv, idx_vv, redbuf, outrow, shared):
        c = lax.axis_index("c")
        s = lax.axis_index("s")
        pltpu.sync_copy(idx_hbm, idx_vv.at[pl.ds(0, sc_n)])

        def ann_body(a, _):
            j = 2 * a + c
            row = idx_vv[pl.ds(j, 16)][0]
            pltpu.sync_copy(feats_hbm.at[row, s], slab)
            pltpu.sync_copy(wn_hbm.at[pl.ds(j * tok, tok)], w_vv.at[pl.ds(0, tok)])

            def chunk_body(q, _):
                def tok_body(n, accs):
                    wv = w_vv[pl.ds(tpt * s + n, 16)][0]
                    return tuple(
                        accs[r] + wv * slab[n, pl.ds(64 * q + 16 * r, 16)]
                        for r in range(4))
                accs = lax.fori_loop(
                    0, tpt, tok_body,
                    tuple(jnp.zeros((16,), jnp.float32) for _ in range(4)))
                for r in range(4):
                    locmf[a, pl.ds(64 * q + 16 * r, 16)] = accs[r]
                return 0
            lax.fori_loop(0, d_mm // 64, chunk_body, 0)
            return 0
        lax.fori_loop(0, n_per_core, ann_body, 0)

        pltpu.sync_copy(locmf, shared.at[s])
        plsc.subcore_barrier()
        pltpu.sync_copy(shared.at[pl.ds(0, 16), s], redbuf)

        def red_chunk(q, _):
            def red_row(r, accs):
                return tuple(
                    accs[t] + redbuf[r, pl.ds(64 * q + 16 * t, 16)]
                    for t in range(4))
            accs = lax.fori_loop(
                0, 16, red_row,
                tuple(jnp.zeros((16,), jnp.float32) for _ in range(4)))
            for t in range(4):
                outrow[pl.ds(64 * q + 16 * t, 16)] = accs[t]
            return 0
        lax.fori_loop(0, d_mm // 64, red_chunk, 0)
        pltpu.sync_copy(outrow, out_hbm.at[2 * s + c])

    return sc_pool


def _mlp1_body(mf_ref, w1_ref, b1_ref, h_ref):
    h = jax.lax.dot_general(
        mf_ref[...], w1_ref[...], (((1,), (1,)), ((), ())),
        preferred_element_type=jnp.float32)
    h = h + b1_ref[...]
    h_ref[...] = 0.5 * h * (1.0 + jax.lax.erf(h * _INV_SQRT2))


def _mlp2_body(h_ref, w2a_ref, w2b_ref, b2_ref, o_ref):
    half = w2a_ref.shape[0]
    for g, w2_ref in enumerate((w2a_ref, w2b_ref)):
        o = jax.lax.dot_general(
            h_ref[...], w2_ref[...], (((1,), (1,)), ((), ())),
            preferred_element_type=jnp.float32)
        o_ref[:, g * half:(g + 1) * half] = o + b2_ref[:, g * half:(g + 1) * half]


def kernel(feats, masks, X_features, ann_indices, frame_nums, W1, b1, W2, b2):
    del X_features, frame_nums
    num_imgs, n_ann = masks.shape[0], masks.shape[1]
    total = num_imgs * n_ann
    tok, d_mm = feats.shape[1], feats.shape[2]
    d_hid = W1.shape[0]

    masks3 = masks.reshape(total, _H, _H)
    sup = jnp.asarray(_SUP)
    supt = jnp.asarray(_SUP.T)

    w = pl.pallas_call(
        functools.partial(_weights_body, n_ann=n_ann),
        grid=(num_imgs,),
        in_specs=[
            pl.BlockSpec((n_ann // 2, _H, _H), lambda i: (2 * i, 0, 0)),
            pl.BlockSpec((n_ann // 2, _H, _H), lambda i: (2 * i + 1, 0, 0)),
            pl.BlockSpec((_NG, _H), lambda i: (0, 0)),
            pl.BlockSpec((_H, _NG), lambda i: (0, 0)),
        ],
        out_specs=pl.BlockSpec((n_ann, _NG, _NG), lambda i: (i, 0, 0)),
        out_shape=jax.ShapeDtypeStruct((total, _NG, _NG), jnp.float32),
    )(masks3, masks3, sup, supt)
    w3 = w.reshape(total, 1, tok)

    ann_flat = ann_indices.reshape(-1).astype(jnp.int32)
    sc_n = 32
    tc_n = total - sc_n
    grid_spec = pltpu.PrefetchScalarGridSpec(
        num_scalar_prefetch=1,
        grid=(tc_n // _NSTREAM,),
        in_specs=[pl.BlockSpec((_NSTREAM, 1, tok), lambda a, idx: (a, 0, 0))] + [
            pl.BlockSpec((1, tok, d_mm),
                         functools.partial(lambda a, idx, k=0: (idx[_NSTREAM * a + k], 0, 0), k=k))
            for k in range(_NSTREAM)
        ],
        out_specs=pl.BlockSpec((_NSTREAM, 1, d_mm), lambda a, idx: (a, 0, 0)),
    )
    mf_tc = pl.pallas_call(
        _pool_body,
        grid_spec=grid_spec,
        out_shape=jax.ShapeDtypeStruct((tc_n, 1, d_mm), jnp.float32),
    )(ann_flat[:tc_n], w3[:tc_n], *([feats] * _NSTREAM)).reshape(tc_n, d_mm)

    wn_flat = w.reshape(total, tok)
    feats4 = feats.reshape(feats.shape[0], 16, tok // 16, d_mm)
    mf_sc = _sc_pool_factory(sc_n, tok, d_mm)(
        feats4, wn_flat[tc_n:].reshape(-1), ann_flat[tc_n:])
    mask_feat = jnp.concatenate([mf_tc, mf_sc], axis=0)

    h = pl.pallas_call(
        _mlp1_body,
        in_specs=[
            pl.BlockSpec((total, d_mm), lambda: (0, 0)),
            pl.BlockSpec((d_hid, d_mm), lambda: (0, 0)),
            pl.BlockSpec((1, d_hid), lambda: (0, 0)),
        ],
        out_specs=pl.BlockSpec((total, d_hid), lambda: (0, 0)),
        out_shape=jax.ShapeDtypeStruct((total, d_hid), jnp.float32),
    )(mask_feat, W1, b1.reshape(1, d_hid))

    n_chunks = 8
    chunk = d_hid // n_chunks
    out = pl.pallas_call(
        _mlp2_body,
        grid=(n_chunks,),
        in_specs=[
            pl.BlockSpec((total, d_hid), lambda n: (0, 0)),
            pl.BlockSpec((chunk // 2, d_hid), lambda n: (2 * n, 0)),
            pl.BlockSpec((chunk // 2, d_hid), lambda n: (2 * n + 1, 0)),
            pl.BlockSpec((1, chunk), lambda n: (0, n)),
        ],
        out_specs=pl.BlockSpec((total, chunk), lambda n: (0, n)),
        out_shape=jax.ShapeDtypeStruct((total, d_hid), jnp.float32),
    )(h, W2, W2, b2.reshape(1, d_hid))

    region_token_nums = jnp.full(
        (num_imgs * ann_indices.shape[1],), ann_indices.shape[2], jnp.int32)
    return (out, region_token_nums)


# speedup vs baseline: 4.1797x; 4.1797x over previous
import functools

import numpy as np
import jax
import jax.numpy as jnp
from jax.experimental import pallas as pl
from jax.experimental.pallas import tpu as pltpu

_H = 336
_NG = 24
_SCALE = _H // _NG


def _support_matrix() -> np.ndarray:
    i = np.arange(_NG)[:, None]
    j = np.arange(_H)[None, :]
    return (np.abs(j - (_SCALE * i + (_SCALE / 2 - 0.5))) < _SCALE).astype(np.float32)


_SUP = _support_matrix()
_NSTREAM = 4
_INV_SQRT2 = float(1.0 / np.sqrt(2.0))


def _weights_body(m0_ref, m1_ref, sup_ref, supt_ref, w_ref, *, n_ann):
    half = n_ann // 2
    for g, mask_ref in enumerate((m0_ref, m1_ref)):
        m = mask_ref[...].astype(jnp.float32)
        mf = m.reshape(half * _H, _H)
        colp = jax.lax.dot_general(
            mf, supt_ref[...], (((1,), (0,)), ((), ())),
            preferred_element_type=jnp.float32)
        for a in range(half):
            t = colp[a * _H:(a + 1) * _H]
            q = jax.lax.dot_general(
                sup_ref[...], t, (((1,), (0,)), ((), ())),
                preferred_element_type=jnp.float32)
            w_ref[g * half + a] = (q > 0).astype(jnp.float32)


def _pool_body(idx_ref, w_ref, *refs):
    del idx_ref
    out_ref = refs[-1]
    for k, f_ref in enumerate(refs[:-1]):
        w = w_ref[k]
        s = jnp.sum(w)
        f = f_ref[0]
        mf = jax.lax.dot_general(
            w, f, (((1,), (0,)), ((), ())),
            preferred_element_type=jnp.float32)
        out_ref[k] = mf * (1.0 / (s + 1e-8))


def _mlp1_body(mf_ref, w1_ref, b1_ref, h_ref):
    h = jax.lax.dot_general(
        mf_ref[...], w1_ref[...], (((1,), (1,)), ((), ())),
        preferred_element_type=jnp.float32)
    h = h + b1_ref[...]
    h_ref[...] = 0.5 * h * (1.0 + jax.lax.erf(h * _INV_SQRT2))


def _mlp2_body(h_ref, w2a_ref, w2b_ref, b2_ref, o_ref):
    half = w2a_ref.shape[0]
    for g, w2_ref in enumerate((w2a_ref, w2b_ref)):
        o = jax.lax.dot_general(
            h_ref[...], w2_ref[...], (((1,), (1,)), ((), ())),
            preferred_element_type=jnp.float32)
        o_ref[:, g * half:(g + 1) * half] = o + b2_ref[:, g * half:(g + 1) * half]


def kernel(feats, masks, X_features, ann_indices, frame_nums, W1, b1, W2, b2):
    del X_features, frame_nums
    num_imgs, n_ann = masks.shape[0], masks.shape[1]
    total = num_imgs * n_ann
    tok, d_mm = feats.shape[1], feats.shape[2]
    d_hid = W1.shape[0]

    masks3 = masks.reshape(total, _H, _H)
    sup = jnp.asarray(_SUP)
    supt = jnp.asarray(_SUP.T)

    w = pl.pallas_call(
        functools.partial(_weights_body, n_ann=n_ann),
        grid=(num_imgs,),
        in_specs=[
            pl.BlockSpec((n_ann // 2, _H, _H), lambda i: (2 * i, 0, 0)),
            pl.BlockSpec((n_ann // 2, _H, _H), lambda i: (2 * i + 1, 0, 0)),
            pl.BlockSpec((_NG, _H), lambda i: (0, 0)),
            pl.BlockSpec((_H, _NG), lambda i: (0, 0)),
        ],
        out_specs=pl.BlockSpec((n_ann, _NG, _NG), lambda i: (i, 0, 0)),
        out_shape=jax.ShapeDtypeStruct((total, _NG, _NG), jnp.float32),
    )(masks3, masks3, sup, supt)
    w3 = w.reshape(total, 1, tok)

    ann_flat = ann_indices.reshape(-1).astype(jnp.int32)
    grid_spec = pltpu.PrefetchScalarGridSpec(
        num_scalar_prefetch=1,
        grid=(total // _NSTREAM,),
        in_specs=[pl.BlockSpec((_NSTREAM, 1, tok), lambda a, idx: (a, 0, 0))] + [
            pl.BlockSpec((1, tok, d_mm),
                         functools.partial(lambda a, idx, k=0: (idx[_NSTREAM * a + k], 0, 0), k=k))
            for k in range(_NSTREAM)
        ],
        out_specs=pl.BlockSpec((_NSTREAM, 1, d_mm), lambda a, idx: (a, 0, 0)),
    )
    mask_feat = pl.pallas_call(
        _pool_body,
        grid_spec=grid_spec,
        out_shape=jax.ShapeDtypeStruct((total, 1, d_mm), jnp.float32),
    )(ann_flat, w3, *([feats] * _NSTREAM)).reshape(total, d_mm)

    h = pl.pallas_call(
        _mlp1_body,
        in_specs=[
            pl.BlockSpec((total, d_mm), lambda: (0, 0)),
            pl.BlockSpec((d_hid, d_mm), lambda: (0, 0)),
            pl.BlockSpec((1, d_hid), lambda: (0, 0)),
        ],
        out_specs=pl.BlockSpec((total, d_hid), lambda: (0, 0)),
        out_shape=jax.ShapeDtypeStruct((total, d_hid), jnp.float32),
    )(mask_feat, W1, b1.reshape(1, d_hid))

    n_chunks = 8
    chunk = d_hid // n_chunks
    out = pl.pallas_call(
        _mlp2_body,
        grid=(n_chunks,),
        in_specs=[
            pl.BlockSpec((total, d_hid), lambda n: (0, 0)),
            pl.BlockSpec((chunk // 2, d_hid), lambda n: (2 * n, 0)),
            pl.BlockSpec((chunk // 2, d_hid), lambda n: (2 * n + 1, 0)),
            pl.BlockSpec((1, chunk), lambda n: (0, n)),
        ],
        out_specs=pl.BlockSpec((total, chunk), lambda n: (0, n)),
        out_shape=jax.ShapeDtypeStruct((total, d_hid), jnp.float32),
    )(h, W2, W2, b2.reshape(1, d_hid))

    region_token_nums = jnp.full(
        (num_imgs * ann_indices.shape[1],), ann_indices.shape[2], jnp.int32)
    return (out, region_token_nums)


# speedup vs baseline: 4.2594x; 1.0191x over previous
import functools

import numpy as np
import jax
import jax.numpy as jnp
from jax.experimental import pallas as pl
from jax.experimental.pallas import tpu as pltpu

_H = 336
_NG = 24
_SCALE = _H // _NG


def _support_matrix() -> np.ndarray:
    i = np.arange(_NG)[:, None]
    j = np.arange(_H)[None, :]
    return (np.abs(j - (_SCALE * i + (_SCALE / 2 - 0.5))) < _SCALE).astype(np.float32)


_SUP = _support_matrix()
_NSTREAM = 4
_INV_SQRT2 = float(1.0 / np.sqrt(2.0))


def _weights_body(m0_ref, m1_ref, sup_ref, supt_ref, w_ref, *, n_ann):
    half = n_ann // 2
    for g, mask_ref in enumerate((m0_ref, m1_ref)):
        m = mask_ref[...].astype(jnp.float32)
        mf = m.reshape(half * _H, _H)
        colp = jax.lax.dot_general(
            mf, supt_ref[...], (((1,), (0,)), ((), ())),
            preferred_element_type=jnp.float32)
        for a in range(half):
            t = colp[a * _H:(a + 1) * _H]
            q = jax.lax.dot_general(
                sup_ref[...], t, (((1,), (0,)), ((), ())),
                preferred_element_type=jnp.float32)
            w_ref[g * half + a] = (q > 0).astype(jnp.float32)


def _pool_body(idx_ref, w_ref, *refs):
    del idx_ref
    out_ref = refs[-1]
    for k, f_ref in enumerate(refs[:-1]):
        w = w_ref[k]
        s = jnp.sum(w)
        f = f_ref[0]
        mf = jax.lax.dot_general(
            w, f, (((1,), (0,)), ((), ())),
            preferred_element_type=jnp.float32)
        out_ref[k] = mf * (1.0 / (s + 1e-8))


def _mlp_body(mf_ref, w1_ref, b1_ref, w2a_ref, w2b_ref, b2_ref, o_ref, h_ref):
    @pl.when(pl.program_id(0) == 0)
    def _():
        h = jax.lax.dot_general(
            mf_ref[...], w1_ref[...], (((1,), (1,)), ((), ())),
            preferred_element_type=jnp.float32)
        h = h + b1_ref[...]
        h_ref[...] = 0.5 * h * (1.0 + jax.lax.erf(h * _INV_SQRT2))

    half = w2a_ref.shape[0]
    for g, w2_ref in enumerate((w2a_ref, w2b_ref)):
        o = jax.lax.dot_general(
            h_ref[...], w2_ref[...], (((1,), (1,)), ((), ())),
            preferred_element_type=jnp.float32)
        o_ref[:, g * half:(g + 1) * half] = o + b2_ref[:, g * half:(g + 1) * half]


def kernel(feats, masks, X_features, ann_indices, frame_nums, W1, b1, W2, b2):
    del X_features, frame_nums
    num_imgs, n_ann = masks.shape[0], masks.shape[1]
    total = num_imgs * n_ann
    tok, d_mm = feats.shape[1], feats.shape[2]
    d_hid = W1.shape[0]

    masks3 = masks.reshape(total, _H, _H)
    sup = jnp.asarray(_SUP)
    supt = jnp.asarray(_SUP.T)

    w = pl.pallas_call(
        functools.partial(_weights_body, n_ann=n_ann),
        grid=(num_imgs,),
        in_specs=[
            pl.BlockSpec((n_ann // 2, _H, _H), lambda i: (2 * i, 0, 0)),
            pl.BlockSpec((n_ann // 2, _H, _H), lambda i: (2 * i + 1, 0, 0)),
            pl.BlockSpec((_NG, _H), lambda i: (0, 0)),
            pl.BlockSpec((_H, _NG), lambda i: (0, 0)),
        ],
        out_specs=pl.BlockSpec((n_ann, _NG, _NG), lambda i: (i, 0, 0)),
        out_shape=jax.ShapeDtypeStruct((total, _NG, _NG), jnp.float32),
    )(masks3, masks3, sup, supt)
    w3 = w.reshape(total, 1, tok)

    ann_flat = ann_indices.reshape(-1).astype(jnp.int32)
    grid_spec = pltpu.PrefetchScalarGridSpec(
        num_scalar_prefetch=1,
        grid=(total // _NSTREAM,),
        in_specs=[pl.BlockSpec((_NSTREAM, 1, tok), lambda a, idx: (a, 0, 0))] + [
            pl.BlockSpec((1, tok, d_mm),
                         functools.partial(lambda a, idx, k=0: (idx[_NSTREAM * a + k], 0, 0), k=k))
            for k in range(_NSTREAM)
        ],
        out_specs=pl.BlockSpec((_NSTREAM, 1, d_mm), lambda a, idx: (a, 0, 0)),
    )
    mask_feat = pl.pallas_call(
        _pool_body,
        grid_spec=grid_spec,
        out_shape=jax.ShapeDtypeStruct((total, 1, d_mm), jnp.float32),
    )(ann_flat, w3, *([feats] * _NSTREAM)).reshape(total, d_mm)

    n_chunks = 8
    chunk = d_hid // n_chunks
    out = pl.pallas_call(
        _mlp_body,
        grid=(n_chunks,),
        in_specs=[
            pl.BlockSpec((total, d_mm), lambda n: (0, 0)),
            pl.BlockSpec((d_hid, d_mm), lambda n: (0, 0)),
            pl.BlockSpec((1, d_hid), lambda n: (0, 0)),
            pl.BlockSpec((chunk // 2, d_hid), lambda n: (2 * n, 0)),
            pl.BlockSpec((chunk // 2, d_hid), lambda n: (2 * n + 1, 0)),
            pl.BlockSpec((1, chunk), lambda n: (0, n)),
        ],
        out_specs=pl.BlockSpec((total, chunk), lambda n: (0, n)),
        out_shape=jax.ShapeDtypeStruct((total, d_hid), jnp.float32),
        scratch_shapes=[pltpu.VMEM((total, d_hid), jnp.float32)],
    )(mask_feat, W1, b1.reshape(1, d_hid), W2, W2, b2.reshape(1, d_hid))

    region_token_nums = jnp.full(
        (num_imgs * ann_indices.shape[1],), ann_indices.shape[2], jnp.int32)
    return (out, region_token_nums)
